# baseline (speedup 1.0000x reference)
import jax
import jax.numpy as jnp
from jax.experimental import pallas as pl
from jax.experimental.pallas import tpu as pltpu

N_TOKENS = 16384
HIDDEN_DIM = 2048
NUM_EXPERTS = 64
TILE = 1024
HALF_TOKENS = N_TOKENS // 2
NSTEPS = HALF_TOKENS // TILE


def _router_kernel(x1_ref, x2_ref, w_ref, b_ref, o_ref):
    w = w_ref[...].astype(jnp.bfloat16)
    bias = b_ref[...]

    def probs(x_f32):
        logits = jax.lax.dot_general(
            x_f32.astype(jnp.bfloat16), w, (((1,), (1,)), ((), ())),
            preferred_element_type=jnp.float32,
        ) + bias
        m = jnp.max(logits, axis=-1, keepdims=True)
        e = jnp.exp(logits - m)
        return e / jnp.sum(e, axis=-1, keepdims=True)

    o_ref[0] = probs(x1_ref[0])
    o_ref[1] = probs(x2_ref[0])


def kernel(x, W, b):
    x3 = x.reshape(2, HALF_TOKENS, HIDDEN_DIM)
    b2 = b.reshape(1, NUM_EXPERTS)
    out = pl.pallas_call(
        _router_kernel,
        grid=(NSTEPS,),
        in_specs=[
            pl.BlockSpec((1, TILE, HIDDEN_DIM), lambda i: (0, i, 0)),
            pl.BlockSpec((1, TILE, HIDDEN_DIM), lambda i: (1, i, 0)),
            pl.BlockSpec((NUM_EXPERTS, HIDDEN_DIM), lambda i: (0, 0)),
            pl.BlockSpec((1, NUM_EXPERTS), lambda i: (0, 0)),
        ],
        out_specs=pl.BlockSpec((2, TILE, NUM_EXPERTS), lambda i: (0, i, 0)),
        out_shape=jax.ShapeDtypeStruct((2, HALF_TOKENS, NUM_EXPERTS), jnp.float32),
        compiler_params=pltpu.CompilerParams(
            dimension_semantics=("arbitrary",),
        ),
    )(x3, x3, W, b2)
    return out.reshape(N_TOKENS, NUM_EXPERTS)


# speedup vs baseline: 1.0166x; 1.0166x over previous
import jax
import jax.numpy as jnp
from jax.experimental import pallas as pl
from jax.experimental.pallas import tpu as pltpu

N_TOKENS = 16384
HIDDEN_DIM = 2048
NUM_EXPERTS = 64
TILE = 512
HALF_TOKENS = N_TOKENS // 2
NSTEPS = HALF_TOKENS // TILE


def _router_kernel(x1_ref, x2_ref, w_ref, b_ref, o_ref):
    w = w_ref[...].astype(jnp.bfloat16)
    bias = b_ref[...]

    def probs(x_f32):
        logits = jax.lax.dot_general(
            x_f32.astype(jnp.bfloat16), w, (((1,), (1,)), ((), ())),
            preferred_element_type=jnp.float32,
        ) + bias
        m = jnp.max(logits, axis=-1, keepdims=True)
        e = jnp.exp(logits - m)
        return e / jnp.sum(e, axis=-1, keepdims=True)

    o_ref[0] = probs(x1_ref[0])
    o_ref[1] = probs(x2_ref[0])


def kernel(x, W, b):
    x3 = x.reshape(2, HALF_TOKENS, HIDDEN_DIM)
    b2 = b.reshape(1, NUM_EXPERTS)
    out = pl.pallas_call(
        _router_kernel,
        grid=(NSTEPS,),
        in_specs=[
            pl.BlockSpec((1, TILE, HIDDEN_DIM), lambda i: (0, i, 0)),
            pl.BlockSpec((1, TILE, HIDDEN_DIM), lambda i: (1, i, 0)),
            pl.BlockSpec((NUM_EXPERTS, HIDDEN_DIM), lambda i: (0, 0)),
            pl.BlockSpec((1, NUM_EXPERTS), lambda i: (0, 0)),
        ],
        out_specs=pl.BlockSpec((2, TILE, NUM_EXPERTS), lambda i: (0, i, 0)),
        out_shape=jax.ShapeDtypeStruct((2, HALF_TOKENS, NUM_EXPERTS), jnp.float32),
        compiler_params=pltpu.CompilerParams(
            dimension_semantics=("arbitrary",),
        ),
    )(x3, x3, W, b2)
    return out.reshape(N_TOKENS, NUM_EXPERTS)
